# baseline (speedup 1.0000x reference)
import jax
import jax.numpy as jnp
from jax import lax
from jax.experimental import pallas as pl
from jax.experimental.pallas import tpu as pltpu
from jax.experimental.pallas import tpu_sc as plsc

N_RAYS = 16384
N_SAMPLES = 64
GRID = 64
S = GRID + 1
SH_DIM = 9
DATA_DIM = 28
DPAD = 32
NEAR = 0.0
FAR = 2.0
BG = 1.0
STEP = (FAR - NEAR) / N_SAMPLES

NC = 2
NS = 16
LANES = 16
NW = NC * NS
RAYS_PER_TILE = N_RAYS // NW
GROUPS = RAYS_PER_TILE // LANES
NBUF = 4
NPAIR = 4
ROWS = NPAIR * LANES
NDIAG = DPAD

_OFFP = [0, S, S * S, S * S + S]

_C0 = 0.28209479177387814
_C1 = 0.4886025119029199
_C20 = 1.0925484305920792
_C21 = -1.0925484305920792
_C22 = 0.31539156525252005
_C23 = -1.0925484305920792
_C24 = 0.5462742152960396


def _body(table_ref, rays_ref, out_ref,
          rayv, idxv, wbuf, rows_bufs, outv,
          cvecs, basisb, maskb, coefb, sems):
  cid = lax.axis_index("c")
  sid = lax.axis_index("s")
  wid = sid * NC + cid

  pltpu.sync_copy(rays_ref.at[wid], rayv)

  lane = lax.iota(jnp.int32, LANES)
  rvecs = [p * LANES + lane for p in range(NPAIR)]

  def mk_tables(u, carry):
    blk = u // LANES
    i = u % LANES
    fvec = blk * LANES + ((i + lane) & (LANES - 1))
    cvecs[u, pl.ds(0, LANES)] = fvec
    kk = fvec // SH_DIM
    for k in range(3):
      maskb[3 * u + k, pl.ds(0, LANES)] = jnp.where(
          kk == k, 1.0, 0.0).astype(jnp.float32)
    coefb[4 * u + 3, pl.ds(0, LANES)] = jnp.where(
        fvec == 3 * SH_DIM, 1.0, 0.0).astype(jnp.float32)
    return carry

  lax.fori_loop(0, NDIAG, mk_tables, 0)

  def start(t, rb, sb):
    pltpu.make_async_copy(table_ref.at[idxv.at[t]], rb, sb).start()

  def wait(t, rb, sb):
    pltpu.make_async_copy(table_ref.at[idxv.at[t]], rb, sb).wait()

  def group_body(g, carry0):
    sl = pl.ds(g * LANES, LANES)
    ox = rayv[0, sl]
    oy = rayv[1, sl]
    oz = rayv[2, sl]
    dx = rayv[3, sl]
    dy = rayv[4, sl]
    dz = rayv[5, sl]
    delta = rayv[6, sl]

    basisb[0, pl.ds(0, LANES)] = jnp.full((LANES,), _C0, jnp.float32)
    basisb[1, pl.ds(0, LANES)] = -_C1 * dy
    basisb[2, pl.ds(0, LANES)] = _C1 * dz
    basisb[3, pl.ds(0, LANES)] = -_C1 * dx
    basisb[4, pl.ds(0, LANES)] = _C20 * dx * dy
    basisb[5, pl.ds(0, LANES)] = _C21 * dy * dz
    basisb[6, pl.ds(0, LANES)] = _C22 * (2.0 * dz * dz - dx * dx - dy * dy)
    basisb[7, pl.ds(0, LANES)] = _C23 * dx * dz
    basisb[8, pl.ds(0, LANES)] = _C24 * (dx * dx - dy * dy)

    def mk_coef(u, carry):
      fvec = cvecs[u, pl.ds(0, LANES)]
      bperm = plsc.load_gather(basisb, [fvec % SH_DIM, lane])
      for k in range(3):
        coefb[4 * u + k, pl.ds(0, LANES)] = (
            bperm * maskb[3 * u + k, pl.ds(0, LANES)])
      return carry

    lax.fori_loop(0, NDIAG, mk_coef, 0)

    def p1(t, carry):
      tt = (t.astype(jnp.float32) + 0.5) * STEP + NEAR
      px = ox + tt * dx
      py = oy + tt * dy
      pz = oz + tt * dz
      posx = jnp.clip(0.5 + 0.5 * px, 0.0, 1.0 - 1e-6) * GRID
      posy = jnp.clip(0.5 + 0.5 * py, 0.0, 1.0 - 1e-6) * GRID
      posz = jnp.clip(0.5 + 0.5 * pz, 0.0, 1.0 - 1e-6) * GRID
      ix = posx.astype(jnp.int32)
      iy = posy.astype(jnp.int32)
      iz = posz.astype(jnp.int32)
      fx = posx - ix.astype(jnp.float32)
      fy = posy - iy.astype(jnp.float32)
      fz = posz - iz.astype(jnp.float32)
      idx000 = (ix * S + iy) * S + iz
      for p in range(NPAIR):
        idxv[t, pl.ds(p * LANES, LANES)] = idx000 + _OFFP[p]
      wbuf[t, pl.ds(0, LANES)] = fx
      wbuf[t, pl.ds(LANES, LANES)] = fy
      wbuf[t, pl.ds(2 * LANES, LANES)] = fz
      return carry

    lax.fori_loop(0, N_SAMPLES, p1, 0)

    for b in range(NBUF):
      start(b, rows_bufs[b], sems[b])

    def p2(tq, carry):
      tr, aw, rr, rg, rb_ = carry
      for b in range(NBUF):
        rbuf = rows_bufs[b]
        sbuf = sems[b]
        t = NBUF * tq + b
        wait(t, rbuf, sbuf)
        fx = wbuf[t, pl.ds(0, LANES)]
        fy = wbuf[t, pl.ds(LANES, LANES)]
        fz = wbuf[t, pl.ds(2 * LANES, LANES)]
        wx0 = 1.0 - fx
        wy0 = 1.0 - fy
        wz0 = 1.0 - fz
        wxy = [wx0 * wy0, wx0 * fy, fx * wy0, fx * fy]
        w = []
        for cc in range(8):
          wz = wz0 if (cc & 1) == 0 else fz
          w.append(wxy[cc >> 1] * wz)

        def diag(un, carry2):
          l0, l1, l2, l3 = carry2
          for uu in range(4):
            u = 4 * un + uu
            cvec = cvecs[u, pl.ds(0, LANES)]
            cvecz = cvec + DPAD
            v01 = (w[0] * plsc.load_gather(rbuf, [rvecs[0], cvec])
                   + w[1] * plsc.load_gather(rbuf, [rvecs[0], cvecz]))
            v23 = (w[2] * plsc.load_gather(rbuf, [rvecs[1], cvec])
                   + w[3] * plsc.load_gather(rbuf, [rvecs[1], cvecz]))
            v45 = (w[4] * plsc.load_gather(rbuf, [rvecs[2], cvec])
                   + w[5] * plsc.load_gather(rbuf, [rvecs[2], cvecz]))
            v67 = (w[6] * plsc.load_gather(rbuf, [rvecs[3], cvec])
                   + w[7] * plsc.load_gather(rbuf, [rvecs[3], cvecz]))
            v = (v01 + v23) + (v45 + v67)
            l0 = l0 + v * coefb[4 * u, pl.ds(0, LANES)]
            l1 = l1 + v * coefb[4 * u + 1, pl.ds(0, LANES)]
            l2 = l2 + v * coefb[4 * u + 2, pl.ds(0, LANES)]
            l3 = l3 + v * coefb[4 * u + 3, pl.ds(0, LANES)]
          return (l0, l1, l2, l3)

        z16 = jnp.zeros((LANES,), jnp.float32)
        l0, l1, l2, l3 = lax.fori_loop(
            0, NDIAG // 4, diag, (z16, z16, z16, z16))

        sig = jnp.maximum(l3, 0.0)
        c0 = 1.0 / (1.0 + jnp.exp(-l0))
        c1 = 1.0 / (1.0 + jnp.exp(-l1))
        c2 = 1.0 / (1.0 + jnp.exp(-l2))
        alpha = 1.0 - jnp.exp(-sig * delta)
        wgt = alpha * tr
        rr = rr + wgt * c0
        rg = rg + wgt * c1
        rb_ = rb_ + wgt * c2
        aw = aw + wgt
        tr = tr * (1.0 - alpha + 1e-10)

        @pl.when(t + NBUF < N_SAMPLES)
        def _():
          start(t + NBUF, rbuf, sbuf)

      return (tr, aw, rr, rg, rb_)

    ones = jnp.ones((LANES,), jnp.float32)
    zeros = jnp.zeros((LANES,), jnp.float32)
    tr, aw, rr, rg, rb_ = lax.fori_loop(
        0, N_SAMPLES // NBUF, p2, (ones, zeros, zeros, zeros, zeros))
    outv[0, sl] = rr + BG * (1.0 - aw)
    outv[1, sl] = rg + BG * (1.0 - aw)
    outv[2, sl] = rb_ + BG * (1.0 - aw)
    outv[3, sl] = aw
    return carry0

  lax.fori_loop(0, GROUPS, group_body, 0)
  pltpu.sync_copy(outv, out_ref.at[wid])


def _entry(table_ref, rays_ref, out_ref,
           rayv, idxv, wbuf, r0, r1, r2, r3, outv,
           cvecs, basisb, maskb, coefb, s0, s1, s2, s3):
  _body(table_ref, rays_ref, out_ref, rayv, idxv, wbuf,
        (r0, r1, r2, r3), outv, cvecs, basisb, maskb, coefb,
        (s0, s1, s2, s3))


@jax.jit
def kernel(rays_o, rays_d, data):
  norm = jnp.linalg.norm(rays_d, axis=-1, keepdims=True)
  dn = rays_d / (norm + 1e-9)
  delta = STEP * norm
  pad = jnp.zeros((N_RAYS, 1), jnp.float32)
  rd = jnp.concatenate([rays_o, dn, delta, pad], axis=1)
  rays_packed = rd.T.reshape(8, NW, RAYS_PER_TILE).transpose(1, 0, 2)
  tp = jnp.pad(data, ((0, 0), (0, DPAD - DATA_DIM)))
  table = jnp.concatenate([tp[:-1], tp[1:]], axis=1)

  mesh = plsc.VectorSubcoreMesh(
      core_axis_name="c", subcore_axis_name="s",
      num_cores=NC, num_subcores=NS)
  run = pl.kernel(
      _entry,
      out_type=jax.ShapeDtypeStruct((NW, 4, RAYS_PER_TILE), jnp.float32),
      mesh=mesh,
      scratch_types=[
          pltpu.VMEM((8, RAYS_PER_TILE), jnp.float32),
          pltpu.VMEM((N_SAMPLES, ROWS), jnp.int32),
          pltpu.VMEM((N_SAMPLES, 3 * LANES), jnp.float32),
          pltpu.VMEM((ROWS, 2 * DPAD), jnp.float32),
          pltpu.VMEM((ROWS, 2 * DPAD), jnp.float32),
          pltpu.VMEM((ROWS, 2 * DPAD), jnp.float32),
          pltpu.VMEM((ROWS, 2 * DPAD), jnp.float32),
          pltpu.VMEM((4, RAYS_PER_TILE), jnp.float32),
          pltpu.VMEM((NDIAG, LANES), jnp.int32),
          pltpu.VMEM((SH_DIM, LANES), jnp.float32),
          pltpu.VMEM((3 * NDIAG, LANES), jnp.float32),
          pltpu.VMEM((4 * NDIAG, LANES), jnp.float32),
          pltpu.SemaphoreType.DMA,
          pltpu.SemaphoreType.DMA,
          pltpu.SemaphoreType.DMA,
          pltpu.SemaphoreType.DMA,
      ],
      compiler_params=pltpu.CompilerParams(
          needs_layout_passes=False, use_tc_tiling_on_sc=False),
  )
  out = run(table, rays_packed)
  return out.transpose(0, 2, 1).reshape(N_RAYS, 4)[:, :3]


# speedup vs baseline: 1.1784x; 1.1784x over previous
import jax
import jax.numpy as jnp
from jax import lax
from jax.experimental import pallas as pl
from jax.experimental.pallas import tpu as pltpu
from jax.experimental.pallas import tpu_sc as plsc

N_RAYS = 16384
N_SAMPLES = 64
GRID = 64
S = GRID + 1
SH_DIM = 9
DATA_DIM = 28
DPAD = 32
WORDS = DPAD // 2
NEAR = 0.0
FAR = 2.0
BG = 1.0
STEP = (FAR - NEAR) / N_SAMPLES

NC = 2
NS = 16
LANES = 16
NW = NC * NS
RAYS_PER_TILE = N_RAYS // NW
GROUPS = RAYS_PER_TILE // LANES
NBUF = 4
NPAIR = 4
ROWS = NPAIR * LANES
NDIAG = WORDS

_OFFP = [0, S, S * S, S * S + S]

_C0 = 0.28209479177387814
_C1 = 0.4886025119029199
_C20 = 1.0925484305920792
_C21 = -1.0925484305920792
_C22 = 0.31539156525252005
_C23 = -1.0925484305920792
_C24 = 0.5462742152960396


def _body(table_ref, rays_ref, out_ref,
          rayv, idxv, wbuf, rows_bufs, outv,
          cvecs, basisb, maskb, coefb, sems):
  cid = lax.axis_index("c")
  sid = lax.axis_index("s")
  wid = sid * NC + cid

  pltpu.sync_copy(rays_ref.at[wid], rayv)

  lane = lax.iota(jnp.int32, LANES)
  rvecs = [p * LANES + lane for p in range(NPAIR)]

  def mk_tables(u, carry):
    wvec = (u + lane) & (LANES - 1)
    cvecs[u, pl.ds(0, LANES)] = wvec
    fe = 2 * wvec
    fo = fe + 1
    ke = fe // SH_DIM
    ko = fo // SH_DIM
    for k in range(3):
      maskb[6 * u + 2 * k, pl.ds(0, LANES)] = jnp.where(
          ke == k, 1.0, 0.0).astype(jnp.float32)
      maskb[6 * u + 2 * k + 1, pl.ds(0, LANES)] = jnp.where(
          ko == k, 1.0, 0.0).astype(jnp.float32)
    coefb[8 * u + 6, pl.ds(0, LANES)] = jnp.where(
        fe == 3 * SH_DIM, 1.0, 0.0).astype(jnp.float32)
    coefb[8 * u + 7, pl.ds(0, LANES)] = jnp.where(
        fo == 3 * SH_DIM, 1.0, 0.0).astype(jnp.float32)
    return carry

  lax.fori_loop(0, NDIAG, mk_tables, 0)

  def start(t, rb, sb):
    pltpu.make_async_copy(table_ref.at[idxv.at[t]], rb, sb).start()

  def wait(t, rb, sb):
    pltpu.make_async_copy(table_ref.at[idxv.at[t]], rb, sb).wait()

  def group_body(g, carry0):
    sl = pl.ds(g * LANES, LANES)
    ox = rayv[0, sl]
    oy = rayv[1, sl]
    oz = rayv[2, sl]
    dx = rayv[3, sl]
    dy = rayv[4, sl]
    dz = rayv[5, sl]
    delta = rayv[6, sl]

    basisb[0, pl.ds(0, LANES)] = jnp.full((LANES,), _C0, jnp.float32)
    basisb[1, pl.ds(0, LANES)] = -_C1 * dy
    basisb[2, pl.ds(0, LANES)] = _C1 * dz
    basisb[3, pl.ds(0, LANES)] = -_C1 * dx
    basisb[4, pl.ds(0, LANES)] = _C20 * dx * dy
    basisb[5, pl.ds(0, LANES)] = _C21 * dy * dz
    basisb[6, pl.ds(0, LANES)] = _C22 * (2.0 * dz * dz - dx * dx - dy * dy)
    basisb[7, pl.ds(0, LANES)] = _C23 * dx * dz
    basisb[8, pl.ds(0, LANES)] = _C24 * (dx * dx - dy * dy)

    def mk_coef(u, carry):
      wvec = cvecs[u, pl.ds(0, LANES)]
      fe = 2 * wvec
      fo = fe + 1
      bpe = plsc.load_gather(basisb, [fe % SH_DIM, lane])
      bpo = plsc.load_gather(basisb, [fo % SH_DIM, lane])
      for k in range(3):
        coefb[8 * u + 2 * k, pl.ds(0, LANES)] = (
            bpe * maskb[6 * u + 2 * k, pl.ds(0, LANES)])
        coefb[8 * u + 2 * k + 1, pl.ds(0, LANES)] = (
            bpo * maskb[6 * u + 2 * k + 1, pl.ds(0, LANES)])
      return carry

    lax.fori_loop(0, NDIAG, mk_coef, 0)

    def p1(t, carry):
      tt = (t.astype(jnp.float32) + 0.5) * STEP + NEAR
      px = ox + tt * dx
      py = oy + tt * dy
      pz = oz + tt * dz
      posx = jnp.clip(0.5 + 0.5 * px, 0.0, 1.0 - 1e-6) * GRID
      posy = jnp.clip(0.5 + 0.5 * py, 0.0, 1.0 - 1e-6) * GRID
      posz = jnp.clip(0.5 + 0.5 * pz, 0.0, 1.0 - 1e-6) * GRID
      ix = posx.astype(jnp.int32)
      iy = posy.astype(jnp.int32)
      iz = posz.astype(jnp.int32)
      fx = posx - ix.astype(jnp.float32)
      fy = posy - iy.astype(jnp.float32)
      fz = posz - iz.astype(jnp.float32)
      idx000 = (ix * S + iy) * S + iz
      for p in range(NPAIR):
        idxv[t, pl.ds(p * LANES, LANES)] = idx000 + _OFFP[p]
      wbuf[t, pl.ds(0, LANES)] = fx
      wbuf[t, pl.ds(LANES, LANES)] = fy
      wbuf[t, pl.ds(2 * LANES, LANES)] = fz
      return carry

    lax.fori_loop(0, N_SAMPLES, p1, 0)

    for b in range(NBUF):
      start(b, rows_bufs[b], sems[b])

    def p2(tq, carry):
      tr, aw, rr, rg, rb_ = carry
      for b in range(NBUF):
        rbuf = rows_bufs[b]
        sbuf = sems[b]
        t = NBUF * tq + b
        wait(t, rbuf, sbuf)
        fx = wbuf[t, pl.ds(0, LANES)]
        fy = wbuf[t, pl.ds(LANES, LANES)]
        fz = wbuf[t, pl.ds(2 * LANES, LANES)]
        wx0 = 1.0 - fx
        wy0 = 1.0 - fy
        wz0 = 1.0 - fz
        wxy = [wx0 * wy0, wx0 * fy, fx * wy0, fx * fy]
        w = []
        for cc in range(8):
          wz = wz0 if (cc & 1) == 0 else fz
          w.append(wxy[cc >> 1] * wz)
        wp = [plsc.pack(wc, wc, format=plsc.PackFormat.INTERLEAVED)
              for wc in w]

        def diag(un, carry2):
          l0, l1, l2, l3 = carry2
          for uu in range(4):
            u = 4 * un + uu
            cvec = cvecs[u, pl.ds(0, LANES)]
            cvecz = cvec + WORDS
            t0 = (wp[0] * plsc.bitcast(
                      plsc.load_gather(rbuf, [rvecs[0], cvec]), jnp.bfloat16)
                  + wp[1] * plsc.bitcast(
                      plsc.load_gather(rbuf, [rvecs[0], cvecz]), jnp.bfloat16))
            t1 = (wp[2] * plsc.bitcast(
                      plsc.load_gather(rbuf, [rvecs[1], cvec]), jnp.bfloat16)
                  + wp[3] * plsc.bitcast(
                      plsc.load_gather(rbuf, [rvecs[1], cvecz]), jnp.bfloat16))
            t2 = (wp[4] * plsc.bitcast(
                      plsc.load_gather(rbuf, [rvecs[2], cvec]), jnp.bfloat16)
                  + wp[5] * plsc.bitcast(
                      plsc.load_gather(rbuf, [rvecs[2], cvecz]), jnp.bfloat16))
            t3 = (wp[6] * plsc.bitcast(
                      plsc.load_gather(rbuf, [rvecs[3], cvec]), jnp.bfloat16)
                  + wp[7] * plsc.bitcast(
                      plsc.load_gather(rbuf, [rvecs[3], cvecz]), jnp.bfloat16))
            g32 = (t0 + t1) + (t2 + t3)
            ge, go = plsc.unpack(g32, format=plsc.PackFormat.INTERLEAVED)
            l0 = (l0 + ge * coefb[8 * u, pl.ds(0, LANES)]
                  + go * coefb[8 * u + 1, pl.ds(0, LANES)])
            l1 = (l1 + ge * coefb[8 * u + 2, pl.ds(0, LANES)]
                  + go * coefb[8 * u + 3, pl.ds(0, LANES)])
            l2 = (l2 + ge * coefb[8 * u + 4, pl.ds(0, LANES)]
                  + go * coefb[8 * u + 5, pl.ds(0, LANES)])
            l3 = (l3 + ge * coefb[8 * u + 6, pl.ds(0, LANES)]
                  + go * coefb[8 * u + 7, pl.ds(0, LANES)])
          return (l0, l1, l2, l3)

        z16 = jnp.zeros((LANES,), jnp.float32)
        l0, l1, l2, l3 = lax.fori_loop(
            0, NDIAG // 4, diag, (z16, z16, z16, z16))

        sig = jnp.maximum(l3, 0.0)
        c0 = 1.0 / (1.0 + jnp.exp(-l0))
        c1 = 1.0 / (1.0 + jnp.exp(-l1))
        c2 = 1.0 / (1.0 + jnp.exp(-l2))
        alpha = 1.0 - jnp.exp(-sig * delta)
        wgt = alpha * tr
        rr = rr + wgt * c0
        rg = rg + wgt * c1
        rb_ = rb_ + wgt * c2
        aw = aw + wgt
        tr = tr * (1.0 - alpha + 1e-10)

        @pl.when(t + NBUF < N_SAMPLES)
        def _():
          start(t + NBUF, rbuf, sbuf)

      return (tr, aw, rr, rg, rb_)

    ones = jnp.ones((LANES,), jnp.float32)
    zeros = jnp.zeros((LANES,), jnp.float32)
    tr, aw, rr, rg, rb_ = lax.fori_loop(
        0, N_SAMPLES // NBUF, p2, (ones, zeros, zeros, zeros, zeros))
    outv[0, sl] = rr + BG * (1.0 - aw)
    outv[1, sl] = rg + BG * (1.0 - aw)
    outv[2, sl] = rb_ + BG * (1.0 - aw)
    outv[3, sl] = aw
    return carry0

  lax.fori_loop(0, GROUPS, group_body, 0)
  pltpu.sync_copy(outv, out_ref.at[wid])


def _entry(table_ref, rays_ref, out_ref,
           rayv, idxv, wbuf, r0, r1, r2, r3, outv,
           cvecs, basisb, maskb, coefb, s0, s1, s2, s3):
  _body(table_ref, rays_ref, out_ref, rayv, idxv, wbuf,
        (r0, r1, r2, r3), outv, cvecs, basisb, maskb, coefb,
        (s0, s1, s2, s3))


@jax.jit
def kernel(rays_o, rays_d, data):
  norm = jnp.linalg.norm(rays_d, axis=-1, keepdims=True)
  dn = rays_d / (norm + 1e-9)
  delta = STEP * norm
  pad = jnp.zeros((N_RAYS, 1), jnp.float32)
  rd = jnp.concatenate([rays_o, dn, delta, pad], axis=1)
  rays_packed = rd.T.reshape(8, NW, RAYS_PER_TILE).transpose(1, 0, 2)
  db = jnp.pad(data.astype(jnp.bfloat16), ((0, 0), (0, DPAD - DATA_DIM)))
  words = lax.bitcast_convert_type(
      db.reshape(-1, WORDS, 2), jnp.int32)
  table = jnp.concatenate([words[:-1], words[1:]], axis=1)

  mesh = plsc.VectorSubcoreMesh(
      core_axis_name="c", subcore_axis_name="s",
      num_cores=NC, num_subcores=NS)
  run = pl.kernel(
      _entry,
      out_type=jax.ShapeDtypeStruct((NW, 4, RAYS_PER_TILE), jnp.float32),
      mesh=mesh,
      scratch_types=[
          pltpu.VMEM((8, RAYS_PER_TILE), jnp.float32),
          pltpu.VMEM((N_SAMPLES, ROWS), jnp.int32),
          pltpu.VMEM((N_SAMPLES, 3 * LANES), jnp.float32),
          pltpu.VMEM((ROWS, 2 * WORDS), jnp.int32),
          pltpu.VMEM((ROWS, 2 * WORDS), jnp.int32),
          pltpu.VMEM((ROWS, 2 * WORDS), jnp.int32),
          pltpu.VMEM((ROWS, 2 * WORDS), jnp.int32),
          pltpu.VMEM((4, RAYS_PER_TILE), jnp.float32),
          pltpu.VMEM((NDIAG, LANES), jnp.int32),
          pltpu.VMEM((SH_DIM, LANES), jnp.float32),
          pltpu.VMEM((6 * NDIAG, LANES), jnp.float32),
          pltpu.VMEM((8 * NDIAG, LANES), jnp.float32),
          pltpu.SemaphoreType.DMA,
          pltpu.SemaphoreType.DMA,
          pltpu.SemaphoreType.DMA,
          pltpu.SemaphoreType.DMA,
      ],
      compiler_params=pltpu.CompilerParams(
          needs_layout_passes=False, use_tc_tiling_on_sc=False),
  )
  out = run(table, rays_packed)
  return out.transpose(0, 2, 1).reshape(N_RAYS, 4)[:, :3]
